# baseline (speedup 1.0000x reference)
import functools

import jax
import jax.numpy as jnp
from jax import lax
from jax.experimental import pallas as pl
from jax.experimental.pallas import tpu as pltpu
from jax.experimental.pallas import tpu_sc as plsc

NP = 10240
EP = 327680
NC = 2
NS = 16
CHUNK = 128
EDGES_PER_TILE = EP // (NC * NS)
CHUNKS_PER_TILE = EDGES_PER_TILE // CHUNK
ROWS_PER_TILE = NP // NS
RCH = 160
D = 16
NB = 4

_mesh = plsc.VectorSubcoreMesh(
    core_axis_name="c", subcore_axis_name="s", num_cores=NC, num_subcores=NS
)


@functools.partial(
    pl.kernel,
    out_type=jax.ShapeDtypeStruct((NC, NP, D), jnp.float32),
    mesh=_mesh,
    scratch_types=[
        pltpu.VMEM((CHUNKS_PER_TILE, CHUNK), jnp.int32),
        pltpu.VMEM((CHUNK, D), jnp.float32),
        pltpu.VMEM((RCH, D), jnp.float32),
        pltpu.VMEM_SHARED((NP, D), jnp.float32),
        pltpu.SemaphoreType.DMA,
    ],
    compiler_params=pltpu.CompilerParams(use_tc_tiling_on_sc=False),
)
def _deg_kernel(col3_hbm, out_hbm, colv, ones_v, zv, acc, sem):
    cid = lax.axis_index("c")
    sid = lax.axis_index("s")
    wid = cid * NS + sid

    def obody(i, _):
        ones_v[i, :] = jnp.ones((D,), jnp.float32)
        return 0
    lax.fori_loop(0, CHUNK, obody, 0)

    def zbody(i, _):
        zv[i, :] = jnp.zeros((D,), jnp.float32)
        return 0
    lax.fori_loop(0, RCH, zbody, 0)

    pltpu.sync_copy(col3_hbm.at[wid], colv)
    for ch in range(ROWS_PER_TILE // RCH):
        base = sid * ROWS_PER_TILE + ch * RCH
        pltpu.sync_copy(zv, acc.at[pl.ds(base, RCH)])
    plsc.subcore_barrier()

    def gbody(g, _):
        for b in range(8):
            pltpu.async_copy(ones_v, acc.at[colv.at[g * 8 + b]], sem, add=True)
        for b in range(8):
            pltpu.make_async_copy(ones_v, acc.at[colv.at[g * 8 + b]], sem).wait()
        return 0
    lax.fori_loop(0, CHUNKS_PER_TILE // 8, gbody, 0)

    plsc.subcore_barrier()
    for ch in range(ROWS_PER_TILE // RCH):
        base = sid * ROWS_PER_TILE + ch * RCH
        pltpu.sync_copy(acc.at[pl.ds(base, RCH)], out_hbm.at[cid, pl.ds(base, RCH)])


@functools.partial(
    pl.kernel,
    out_type=jax.ShapeDtypeStruct((NC, NP, D), jnp.float32),
    mesh=_mesh,
    scratch_types=[
        pltpu.VMEM((CHUNKS_PER_TILE, CHUNK), jnp.int32),
        pltpu.VMEM((CHUNKS_PER_TILE, CHUNK), jnp.int32),
        pltpu.VMEM((CHUNK, D), jnp.float32),
        pltpu.VMEM((CHUNK, D), jnp.float32),
        pltpu.VMEM((CHUNK, D), jnp.float32),
        pltpu.VMEM((CHUNK, D), jnp.float32),
        pltpu.VMEM((RCH, D), jnp.float32),
        pltpu.VMEM_SHARED((NP, D), jnp.float32),
        pltpu.SemaphoreType.DMA,
        pltpu.SemaphoreType.DMA,
        pltpu.SemaphoreType.DMA,
        pltpu.SemaphoreType.DMA,
        pltpu.SemaphoreType.DMA,
        pltpu.SemaphoreType.DMA,
        pltpu.SemaphoreType.DMA,
        pltpu.SemaphoreType.DMA,
    ],
    compiler_params=pltpu.CompilerParams(use_tc_tiling_on_sc=False),
)
def _agg_kernel(g_hbm, row3_hbm, col3_hbm, out_hbm,
                rowv, colv, m0, m1, m2, m3, zv, acc,
                sg0, sg1, sg2, sg3, ss0, ss1, ss2, ss3):
    cid = lax.axis_index("c")
    sid = lax.axis_index("s")
    wid = cid * NS + sid

    msgs = (m0, m1, m2, m3)
    sg = (sg0, sg1, sg2, sg3)
    ss = (ss0, ss1, ss2, ss3)

    pltpu.sync_copy(row3_hbm.at[wid], rowv)
    pltpu.sync_copy(col3_hbm.at[wid], colv)

    def zbody(i, _):
        zv[i, :] = jnp.zeros((D,), jnp.float32)
        return 0
    lax.fori_loop(0, RCH, zbody, 0)
    for ch in range(ROWS_PER_TILE // RCH):
        base = sid * ROWS_PER_TILE + ch * RCH
        pltpu.sync_copy(zv, acc.at[pl.ds(base, RCH)])
    plsc.subcore_barrier()

    def start_gather(j, b):
        pltpu.async_copy(g_hbm.at[rowv.at[j]], msgs[b], sg[b])

    def wait_gather(j, b):
        pltpu.make_async_copy(g_hbm.at[rowv.at[j]], msgs[b], sg[b]).wait()

    def start_scatter(j, b):
        pltpu.async_copy(msgs[b], acc.at[colv.at[j]], ss[b], add=True)

    def wait_scatter(j, b):
        pltpu.make_async_copy(msgs[b], acc.at[colv.at[j]], ss[b]).wait()

    start_gather(0, 0)
    start_gather(1, 1)

    def body(k, _):
        for d in range(NB):
            b = d
            bf = (d + 2) % NB
            j = k * NB + d
            jf = j + 2
            wait_gather(j, b)
            start_scatter(j, b)
            if d < 2:
                @pl.when(k > 0)
                def _():
                    wait_scatter(jf - NB, bf)
                start_gather(jf, bf)
            else:
                @pl.when(k < CHUNKS_PER_TILE // NB - 1)
                def _():
                    wait_scatter(jf - NB, bf)
                    start_gather(jf, bf)
        return 0
    lax.fori_loop(0, CHUNKS_PER_TILE // NB, body, 0)

    for b in range(NB):
        wait_scatter(CHUNKS_PER_TILE - NB + b, b)

    plsc.subcore_barrier()
    for ch in range(ROWS_PER_TILE // RCH):
        base = sid * ROWS_PER_TILE + ch * RCH
        pltpu.sync_copy(acc.at[pl.ds(base, RCH)], out_hbm.at[cid, pl.ds(base, RCH)])


_BLK = 1024
_GRID = NP // _BLK


def _tc_a_body(x_ref, w_ref, d_ref, g_ref, dinv_ref):
    deg = d_ref[0] + d_ref[1] + 1.0
    dinv = lax.rsqrt(deg)
    h = lax.dot_general(x_ref[...], w_ref[...], (((1,), (1,)), ((), ())),
                        preferred_element_type=jnp.float32)
    g_ref[...] = h * dinv
    dinv_ref[...] = dinv


def _tc_a(xp, W1, degp):
    return pl.pallas_call(
        _tc_a_body,
        grid=(_GRID,),
        in_specs=[
            pl.BlockSpec((_BLK, 128), lambda i: (i, 0)),
            pl.BlockSpec((D, 128), lambda i: (0, 0)),
            pl.BlockSpec((NC, _BLK, D), lambda i: (0, i, 0)),
        ],
        out_specs=[
            pl.BlockSpec((_BLK, D), lambda i: (i, 0)),
            pl.BlockSpec((_BLK, D), lambda i: (i, 0)),
        ],
        out_shape=[
            jax.ShapeDtypeStruct((NP, D), jnp.float32),
            jax.ShapeDtypeStruct((NP, D), jnp.float32),
        ],
    )(xp, W1, degp)


def _tc_b_body(p_ref, g1_ref, dinv_ref, b1_ref, w2_ref, g2_ref):
    out1 = (p_ref[0] + p_ref[1] + g1_ref[...]) * dinv_ref[...] + b1_ref[0:1, :]
    h2 = jnp.maximum(out1, 0.0)
    h2w = lax.dot_general(h2, w2_ref[...], (((1,), (1,)), ((), ())),
                          preferred_element_type=jnp.float32)
    g2_ref[...] = h2w * dinv_ref[...]


def _tc_b(p, g1, dinv, b1p, W2p):
    return pl.pallas_call(
        _tc_b_body,
        grid=(_GRID,),
        in_specs=[
            pl.BlockSpec((NC, _BLK, D), lambda i: (0, i, 0)),
            pl.BlockSpec((_BLK, D), lambda i: (i, 0)),
            pl.BlockSpec((_BLK, D), lambda i: (i, 0)),
            pl.BlockSpec((8, D), lambda i: (0, 0)),
            pl.BlockSpec((D, D), lambda i: (0, 0)),
        ],
        out_specs=pl.BlockSpec((_BLK, D), lambda i: (i, 0)),
        out_shape=jax.ShapeDtypeStruct((NP, D), jnp.float32),
    )(p, g1, dinv, b1p, W2p)


def _tc_c_body(q_ref, g2_ref, dinv_ref, b2_ref, out_ref):
    out_ref[...] = (q_ref[0] + q_ref[1] + g2_ref[...]) * dinv_ref[...] + b2_ref[0:1, :]


def _tc_c(q, g2, dinv, b2p):
    return pl.pallas_call(
        _tc_c_body,
        grid=(_GRID,),
        in_specs=[
            pl.BlockSpec((NC, _BLK, D), lambda i: (0, i, 0)),
            pl.BlockSpec((_BLK, D), lambda i: (i, 0)),
            pl.BlockSpec((_BLK, D), lambda i: (i, 0)),
            pl.BlockSpec((8, D), lambda i: (0, 0)),
        ],
        out_specs=pl.BlockSpec((_BLK, D), lambda i: (i, 0)),
        out_shape=jax.ShapeDtypeStruct((NP, D), jnp.float32),
    )(q, g2, dinv, b2p)


def kernel(x, edge_index, W1, b1, W2, b2):
    n, e = x.shape[0], edge_index.shape[1]

    row = edge_index[0].astype(jnp.int32)
    col = edge_index[1].astype(jnp.int32)
    rowp = (jnp.zeros((EP,), jnp.int32).at[:e].set(row)
            .reshape(NC * NS, CHUNKS_PER_TILE, CHUNK))
    colp = (jnp.full((EP,), NP - 1, jnp.int32).at[:e].set(col)
            .reshape(NC * NS, CHUNKS_PER_TILE, CHUNK))
    xp = jnp.zeros((NP, 128), jnp.float32).at[:n].set(x)
    W2p = jnp.zeros((D, D), jnp.float32).at[: W2.shape[0]].set(W2)
    b1p = jnp.zeros((8, D), jnp.float32).at[0, :].set(b1)
    b2p = jnp.zeros((8, D), jnp.float32).at[0, : b2.shape[0]].set(b2)

    degp = _deg_kernel(colp)
    g1, dinv = _tc_a(xp, W1, degp)
    p = _agg_kernel(g1, rowp, colp)
    g2 = _tc_b(p, g1, dinv, b1p, W2p)
    q = _agg_kernel(g2, rowp, colp)
    out = _tc_c(q, g2, dinv, b2p)
    return out[:n, : b2.shape[0]]


# speedup vs baseline: 1.0604x; 1.0604x over previous
import functools

import jax
import jax.numpy as jnp
from jax import lax
from jax.experimental import pallas as pl
from jax.experimental.pallas import tpu as pltpu
from jax.experimental.pallas import tpu_sc as plsc

NP = 10240
EP = 327680
NC = 2
NS = 16
CHUNK = 128
EDGES_PER_TILE = EP // (NC * NS)
CHUNKS_PER_TILE = EDGES_PER_TILE // CHUNK
ROWS_PER_TILE = NP // NS
RCH = 160
D = 16
NB = 4

_mesh = plsc.VectorSubcoreMesh(
    core_axis_name="c", subcore_axis_name="s", num_cores=NC, num_subcores=NS
)


@functools.partial(
    pl.kernel,
    out_type=jax.ShapeDtypeStruct((NC, NP, D), jnp.float32),
    mesh=_mesh,
    scratch_types=[
        pltpu.VMEM((CHUNKS_PER_TILE, CHUNK), jnp.int32),
        pltpu.VMEM((CHUNK, D), jnp.float32),
        pltpu.VMEM((RCH, D), jnp.float32),
        pltpu.VMEM_SHARED((NP, D), jnp.float32),
        pltpu.SemaphoreType.DMA,
    ],
    compiler_params=pltpu.CompilerParams(use_tc_tiling_on_sc=False),
)
def _deg_kernel(col3_hbm, out_hbm, colv, ones_v, zv, acc, sem):
    cid = lax.axis_index("c")
    sid = lax.axis_index("s")
    wid = cid * NS + sid

    def obody(i, _):
        ones_v[i, :] = jnp.ones((D,), jnp.float32)
        return 0
    lax.fori_loop(0, CHUNK, obody, 0)

    def zbody(i, _):
        zv[i, :] = jnp.zeros((D,), jnp.float32)
        return 0
    lax.fori_loop(0, RCH, zbody, 0)

    pltpu.sync_copy(col3_hbm.at[wid], colv)
    for ch in range(ROWS_PER_TILE // RCH):
        base = sid * ROWS_PER_TILE + ch * RCH
        pltpu.sync_copy(zv, acc.at[pl.ds(base, RCH)])
    plsc.subcore_barrier()

    def gbody(g, _):
        for b in range(8):
            pltpu.async_copy(ones_v, acc.at[colv.at[g * 8 + b]], sem, add=True)
        for b in range(8):
            pltpu.make_async_copy(ones_v, acc.at[colv.at[g * 8 + b]], sem).wait()
        return 0
    lax.fori_loop(0, CHUNKS_PER_TILE // 8, gbody, 0)

    plsc.subcore_barrier()
    for ch in range(ROWS_PER_TILE // RCH):
        base = sid * ROWS_PER_TILE + ch * RCH
        pltpu.sync_copy(acc.at[pl.ds(base, RCH)], out_hbm.at[cid, pl.ds(base, RCH)])


@functools.partial(
    pl.kernel,
    out_type=jax.ShapeDtypeStruct((NC, NP, D), jnp.float32),
    mesh=_mesh,
    scratch_types=[
        pltpu.VMEM((CHUNKS_PER_TILE, CHUNK), jnp.int32),
        pltpu.VMEM((CHUNKS_PER_TILE, CHUNK), jnp.int32),
        pltpu.VMEM((CHUNK, D), jnp.float32),
        pltpu.VMEM((CHUNK, D), jnp.float32),
        pltpu.VMEM((CHUNK, D), jnp.float32),
        pltpu.VMEM((CHUNK, D), jnp.float32),
        pltpu.VMEM((RCH, D), jnp.float32),
        pltpu.VMEM_SHARED((NP, D), jnp.float32),
        pltpu.SemaphoreType.DMA,
        pltpu.SemaphoreType.DMA,
        pltpu.SemaphoreType.DMA,
        pltpu.SemaphoreType.DMA,
        pltpu.SemaphoreType.DMA,
        pltpu.SemaphoreType.DMA,
        pltpu.SemaphoreType.DMA,
        pltpu.SemaphoreType.DMA,
    ],
    compiler_params=pltpu.CompilerParams(use_tc_tiling_on_sc=False),
)
def _agg_kernel(g_hbm, row3_hbm, col3_hbm, out_hbm,
                rowv, colv, m0, m1, m2, m3, zv, acc,
                sg0, sg1, sg2, sg3, ss0, ss1, ss2, ss3):
    cid = lax.axis_index("c")
    sid = lax.axis_index("s")
    wid = cid * NS + sid

    msgs = (m0, m1, m2, m3)
    sg = (sg0, sg1, sg2, sg3)
    ss = (ss0, ss1, ss2, ss3)

    pltpu.sync_copy(row3_hbm.at[wid], rowv)
    pltpu.sync_copy(col3_hbm.at[wid], colv)

    def zbody(i, _):
        zv[i, :] = jnp.zeros((D,), jnp.float32)
        return 0
    lax.fori_loop(0, RCH, zbody, 0)
    for ch in range(ROWS_PER_TILE // RCH):
        base = sid * ROWS_PER_TILE + ch * RCH
        pltpu.sync_copy(zv, acc.at[pl.ds(base, RCH)])
    plsc.subcore_barrier()

    def start_gather(j, b):
        pltpu.async_copy(g_hbm.at[rowv.at[j]], msgs[b], sg[b])

    def wait_gather(j, b):
        pltpu.make_async_copy(g_hbm.at[rowv.at[j]], msgs[b], sg[b]).wait()

    def start_scatter(j, b):
        pltpu.async_copy(msgs[b], acc.at[colv.at[j]], ss[b], add=True)

    def wait_scatter(j, b):
        pltpu.make_async_copy(msgs[b], acc.at[colv.at[j]], ss[b]).wait()

    start_gather(0, 0)
    start_gather(1, 1)

    def body(k, _):
        for d in range(NB):
            b = d
            bf = (d + 2) % NB
            j = k * NB + d
            jf = j + 2
            wait_gather(j, b)
            start_scatter(j, b)
            if d < 2:
                @pl.when(k > 0)
                def _():
                    wait_scatter(jf - NB, bf)
                start_gather(jf, bf)
            else:
                @pl.when(k < CHUNKS_PER_TILE // NB - 1)
                def _():
                    wait_scatter(jf - NB, bf)
                    start_gather(jf, bf)
        return 0
    lax.fori_loop(0, CHUNKS_PER_TILE // NB, body, 0)

    for b in range(NB):
        wait_scatter(CHUNKS_PER_TILE - NB + b, b)

    plsc.subcore_barrier()
    for ch in range(ROWS_PER_TILE // RCH):
        base = sid * ROWS_PER_TILE + ch * RCH
        pltpu.sync_copy(acc.at[pl.ds(base, RCH)], out_hbm.at[cid, pl.ds(base, RCH)])


_BLK = 1024
_GRID = NP // _BLK


def _tc_a_body(x_ref, w_ref, d_ref, g_ref, dinv_ref):
    deg = d_ref[0] + d_ref[1] + 1.0
    dinv = lax.rsqrt(deg)
    h = lax.dot_general(x_ref[...], w_ref[...], (((1,), (1,)), ((), ())),
                        preferred_element_type=jnp.float32)
    g_ref[...] = h * dinv
    dinv_ref[...] = dinv


def _tc_a(xp, W1, degp):
    return pl.pallas_call(
        _tc_a_body,
        grid=(_GRID,),
        in_specs=[
            pl.BlockSpec((_BLK, 128), lambda i: (i, 0)),
            pl.BlockSpec((D, 128), lambda i: (0, 0)),
            pl.BlockSpec((NC, _BLK, D), lambda i: (0, i, 0)),
        ],
        out_specs=[
            pl.BlockSpec((_BLK, D), lambda i: (i, 0)),
            pl.BlockSpec((_BLK, D), lambda i: (i, 0)),
        ],
        out_shape=[
            jax.ShapeDtypeStruct((NP, D), jnp.float32),
            jax.ShapeDtypeStruct((NP, D), jnp.float32),
        ],
    )(xp, W1, degp)


def _tc_b_body(p_ref, g1_ref, dinv_ref, b1_ref, w2_ref, g2_ref):
    out1 = (p_ref[0] + p_ref[1] + g1_ref[...]) * dinv_ref[...] + b1_ref[0:1, :]
    h2 = jnp.maximum(out1, 0.0)
    h2w = lax.dot_general(h2, w2_ref[...], (((1,), (1,)), ((), ())),
                          preferred_element_type=jnp.float32)
    g2_ref[...] = h2w * dinv_ref[...]


def _tc_b(p, g1, dinv, b1p, W2p):
    return pl.pallas_call(
        _tc_b_body,
        grid=(_GRID,),
        in_specs=[
            pl.BlockSpec((NC, _BLK, D), lambda i: (0, i, 0)),
            pl.BlockSpec((_BLK, D), lambda i: (i, 0)),
            pl.BlockSpec((_BLK, D), lambda i: (i, 0)),
            pl.BlockSpec((8, D), lambda i: (0, 0)),
            pl.BlockSpec((D, D), lambda i: (0, 0)),
        ],
        out_specs=pl.BlockSpec((_BLK, D), lambda i: (i, 0)),
        out_shape=jax.ShapeDtypeStruct((NP, D), jnp.float32),
    )(p, g1, dinv, b1p, W2p)


def _tc_c_body(q_ref, g2_ref, dinv_ref, b2_ref, out_ref):
    out_ref[...] = (q_ref[0] + q_ref[1] + g2_ref[...]) * dinv_ref[...] + b2_ref[0:1, :]


def _tc_c(q, g2, dinv, b2p):
    return pl.pallas_call(
        _tc_c_body,
        grid=(_GRID,),
        in_specs=[
            pl.BlockSpec((NC, _BLK, D), lambda i: (0, i, 0)),
            pl.BlockSpec((_BLK, D), lambda i: (i, 0)),
            pl.BlockSpec((_BLK, D), lambda i: (i, 0)),
            pl.BlockSpec((8, D), lambda i: (0, 0)),
        ],
        out_specs=pl.BlockSpec((_BLK, D), lambda i: (i, 0)),
        out_shape=jax.ShapeDtypeStruct((NP, D), jnp.float32),
    )(q, g2, dinv, b2p)


def kernel(x, edge_index, W1, b1, W2, b2):
    n, e = x.shape[0], edge_index.shape[1]

    row = edge_index[0].astype(jnp.int32)
    col = edge_index[1].astype(jnp.int32)
    pad_dst = n + jax.lax.rem(jnp.arange(EP, dtype=jnp.int32), jnp.int32(NP - n))
    rowp = (jnp.zeros((EP,), jnp.int32).at[:e].set(row)
            .reshape(NC * NS, CHUNKS_PER_TILE, CHUNK))
    colp = (pad_dst.at[:e].set(col)
            .reshape(NC * NS, CHUNKS_PER_TILE, CHUNK))
    xp = jnp.zeros((NP, 128), jnp.float32).at[:n].set(x)
    W2p = jnp.zeros((D, D), jnp.float32).at[: W2.shape[0]].set(W2)
    b1p = jnp.zeros((8, D), jnp.float32).at[0, :].set(b1)
    b2p = jnp.zeros((8, D), jnp.float32).at[0, : b2.shape[0]].set(b2)

    degp = _deg_kernel(colp)
    g1, dinv = _tc_a(xp, W1, degp)
    p = _agg_kernel(g1, rowp, colp)
    g2 = _tc_b(p, g1, dinv, b1p, W2p)
    q = _agg_kernel(g2, rowp, colp)
    out = _tc_c(q, g2, dinv, b2p)
    return out[:n, : b2.shape[0]]
